# baseline (speedup 1.0000x reference)
import jax
import jax.numpy as jnp
from jax import lax
from jax.experimental import pallas as pl
from jax.experimental.pallas import tpu as pltpu
from jax.experimental.pallas import tpu_sc as plsc

SEQ_LEN = 200
BATCH = 4096
EMBED_DIM = 64
TOTAL = SEQ_LEN * BATCH

NC = 2
NS = 16
NW = NC * NS
PER_W = TOTAL // NW

IDX_ROW = 128
N_IDX_ROWS = PER_W // IDX_ROW
GPC = 2
CHUNK = IDX_ROW * GPC
NBUF = 4
NCHUNK = PER_W // CHUNK


def _emb_kernel(idx_hbm, table_hbm, out_hbm, idx_v, rows_v, sem_g, sem_o):
    wid = lax.axis_index("s") * NC + lax.axis_index("c")
    base = wid * PER_W

    pltpu.sync_copy(idx_hbm.at[wid], idx_v)

    def fire_gather(c, s):
        for j in range(GPC):
            pltpu.async_copy(
                table_hbm.at[idx_v.at[c * GPC + j]],
                rows_v.at[s].at[pl.ds(j * IDX_ROW, IDX_ROW)],
                sem_g.at[s],
            )

    def wait_gather(c, s):
        for j in range(GPC):
            pltpu.make_async_copy(
                table_hbm.at[idx_v.at[c * GPC + j]],
                rows_v.at[s].at[pl.ds(j * IDX_ROW, IDX_ROW)],
                sem_g.at[s],
            ).wait()

    def fire_out(c, s):
        pltpu.async_copy(
            rows_v.at[s],
            out_hbm.at[pl.ds(base + c * CHUNK, CHUNK)],
            sem_o.at[s],
        )

    def wait_out(c, s):
        pltpu.make_async_copy(
            rows_v.at[s],
            out_hbm.at[pl.ds(base + c * CHUNK, CHUNK)],
            sem_o.at[s],
        ).wait()

    for s in range(NBUF):
        fire_gather(s, s)

    def body(t, carry):
        c0 = t * NBUF
        for s in range(NBUF):
            c = c0 + s
            wait_gather(c, s)
            fire_out(c, s)
            wait_out(c, s)
            fire_gather(c + NBUF, s)
        return carry

    lax.fori_loop(0, NCHUNK // NBUF - 1, body, 0)

    c0 = NCHUNK - NBUF
    for s in range(NBUF):
        c = c0 + s
        wait_gather(c, s)
        fire_out(c, s)
    for s in range(NBUF):
        wait_out(c0 + s, s)


@jax.jit
def kernel(inp, table):
    idx = inp.reshape(NW, N_IDX_ROWS, IDX_ROW)
    mesh = plsc.VectorSubcoreMesh(core_axis_name="c", subcore_axis_name="s")
    out = pl.kernel(
        _emb_kernel,
        out_type=jax.ShapeDtypeStruct((TOTAL, EMBED_DIM), jnp.float32),
        mesh=mesh,
        scratch_types=[
            pltpu.VMEM((N_IDX_ROWS, IDX_ROW), jnp.int32),
            pltpu.VMEM((NBUF, CHUNK, EMBED_DIM), jnp.float32),
            pltpu.SemaphoreType.DMA((NBUF,)),
            pltpu.SemaphoreType.DMA((NBUF,)),
        ],
        compiler_params=pltpu.CompilerParams(use_tc_tiling_on_sc=False),
    )(idx, table)
    return out.reshape(SEQ_LEN, BATCH, EMBED_DIM)


# speedup vs baseline: 1.0030x; 1.0030x over previous
import jax
import jax.numpy as jnp
from jax import lax
from jax.experimental import pallas as pl
from jax.experimental.pallas import tpu as pltpu
from jax.experimental.pallas import tpu_sc as plsc

SEQ_LEN = 200
BATCH = 4096
EMBED_DIM = 64
TOTAL = SEQ_LEN * BATCH

NC = 2
NS = 16
NW = NC * NS
PER_W = TOTAL // NW

IDX_ROW = 128
N_IDX_ROWS = PER_W // IDX_ROW
CHUNK = IDX_ROW
NBUF = 8
LAG = 2
LEAD = NBUF - LAG
NCHUNK = PER_W // CHUNK


def _emb_kernel(idx_hbm, table_hbm, out_hbm, idx_v, rows_v, sem_g, sem_o):
    wid = lax.axis_index("s") * NC + lax.axis_index("c")
    base = wid * PER_W

    pltpu.sync_copy(idx_hbm.at[wid], idx_v)

    def fire_gather(c, s):
        pltpu.async_copy(
            table_hbm.at[idx_v.at[c]],
            rows_v.at[s],
            sem_g.at[s],
        )

    def wait_gather(c, s):
        pltpu.make_async_copy(
            table_hbm.at[idx_v.at[c]],
            rows_v.at[s],
            sem_g.at[s],
        ).wait()

    def fire_out(c, s):
        pltpu.async_copy(
            rows_v.at[s],
            out_hbm.at[pl.ds(base + c * CHUNK, CHUNK)],
            sem_o.at[s],
        )

    def wait_out(c, s):
        pltpu.make_async_copy(
            rows_v.at[s],
            out_hbm.at[pl.ds(base + c * CHUNK, CHUNK)],
            sem_o.at[s],
        ).wait()


    for c in range(LEAD):
        fire_gather(c, c % NBUF)
    for c in range(LAG):
        wait_gather(c, c % NBUF)
        fire_out(c, c % NBUF)
        fire_gather(c + LEAD, (c + LEAD) % NBUF)

    STEADY = NCHUNK - LEAD - LAG
    GROUPS = STEADY // NBUF

    def body(t, carry):
        c0 = LAG + t * NBUF
        for i in range(NBUF):
            c = c0 + i
            s = (LAG + i) % NBUF
            wait_gather(c, s)
            fire_out(c, s)
            wait_out(c - LAG, (s - LAG) % NBUF)
            fire_gather(c + LEAD, (s - LAG) % NBUF)
        return carry

    lax.fori_loop(0, GROUPS, body, 0)

    for c in range(LAG + GROUPS * NBUF, NCHUNK - LEAD):
        s = c % NBUF
        wait_gather(c, s)
        fire_out(c, s)
        wait_out(c - LAG, (c - LAG) % NBUF)
        fire_gather(c + LEAD, (c - LAG) % NBUF)

    for c in range(NCHUNK - LEAD, NCHUNK):
        s = c % NBUF
        wait_gather(c, s)
        fire_out(c, s)
    for c in range(NCHUNK - LEAD - LAG, NCHUNK):
        wait_out(c, c % NBUF)


@jax.jit
def kernel(inp, table):
    idx = inp.reshape(NW, N_IDX_ROWS, IDX_ROW)
    mesh = plsc.VectorSubcoreMesh(core_axis_name="c", subcore_axis_name="s")
    out = pl.kernel(
        _emb_kernel,
        out_type=jax.ShapeDtypeStruct((TOTAL, EMBED_DIM), jnp.float32),
        mesh=mesh,
        scratch_types=[
            pltpu.VMEM((N_IDX_ROWS, IDX_ROW), jnp.int32),
            pltpu.VMEM((NBUF, CHUNK, EMBED_DIM), jnp.float32),
            pltpu.SemaphoreType.DMA((NBUF,)),
            pltpu.SemaphoreType.DMA((NBUF,)),
        ],
        compiler_params=pltpu.CompilerParams(use_tc_tiling_on_sc=False),
    )(idx, table)
    return out.reshape(SEQ_LEN, BATCH, EMBED_DIM)
